# baseline (speedup 1.0000x reference)
import functools

import jax
import jax.numpy as jnp
from jax import lax
from jax.experimental import pallas as pl
from jax.experimental.pallas import tpu as pltpu
from jax.experimental.pallas import tpu_sc as plsc

N = 10000
E = 320000
D = 128
H = D // 2
NC = 2
NS = 16
NW = NC * NS
CH = 80
NCHD = E // NW // CH
CHP = 128
EPT = 20480
NCH = EPT // CHP
NB = 5
L = 2
RB = 624
ZB = 104
BN = 1000


def _sc_mesh():
    return plsc.VectorSubcoreMesh(
        core_axis_name="c", subcore_axis_name="s", num_cores=NC, num_subcores=NS
    )


def _deg_call(dst2d, zeros1d, ones1d):
    @functools.partial(
        pl.kernel,
        out_type=(
            jax.ShapeDtypeStruct((N,), jnp.float32),
            jax.ShapeDtypeStruct((N,), jnp.float32),
        ),
        mesh=_sc_mesh(),
        scratch_types=[
            pltpu.VMEM((NCHD, CH), jnp.int32),
            pltpu.VMEM((CH,), jnp.float32),
            pltpu.VMEM((RB + 16,), jnp.float32),
            pltpu.VMEM_SHARED((N,), jnp.float32),
        ],
    )
    def k(dst_hbm, zero_hbm, ones_hbm, out_a, out_b, idx_d, ones_v, zbuf, acc):
        cid = lax.axis_index("c")
        sid = lax.axis_index("s")
        wid = cid * NS + sid
        pltpu.sync_copy(zero_hbm, zbuf)
        pltpu.sync_copy(zbuf.at[pl.ds(0, RB)], acc.at[pl.ds(sid * RB, RB)])

        @pl.when(sid == NS - 1)
        def _():
            pltpu.sync_copy(zbuf.at[pl.ds(0, 16)], acc.at[pl.ds(NS * RB, 16)])

        pltpu.sync_copy(ones_hbm, ones_v)
        pltpu.sync_copy(dst_hbm.at[wid], idx_d)
        plsc.subcore_barrier()

        def body(j, carry):
            pltpu.sync_copy(ones_v, acc.at[idx_d.at[j]], add=True)
            return carry

        lax.fori_loop(0, NCHD, body, 0)
        plsc.subcore_barrier()

        pltpu.sync_copy(acc.at[pl.ds(sid * RB, RB)], zbuf.at[pl.ds(0, RB)])
        out = [out_a, out_b]
        for c in range(NC):

            @pl.when(cid == c)
            def _(c=c):
                pltpu.sync_copy(zbuf.at[pl.ds(0, RB)], out[c].at[pl.ds(sid * RB, RB)])

                @pl.when(sid == NS - 1)
                def _():
                    pltpu.sync_copy(acc.at[pl.ds(NS * RB, 16)], zbuf.at[pl.ds(RB, 16)])
                    pltpu.sync_copy(zbuf.at[pl.ds(RB, 16)], out[c].at[pl.ds(NS * RB, 16)])

    return k(dst2d, zeros1d, ones1d)


def _agg_call(hs_a, hs_b, src3, dst3, zeros2d):
    @functools.partial(
        pl.kernel,
        out_type=(
            jax.ShapeDtypeStruct((N, H), jnp.float32),
            jax.ShapeDtypeStruct((N, H), jnp.float32),
        ),
        mesh=_sc_mesh(),
        compiler_params=pltpu.CompilerParams(use_tc_tiling_on_sc=False),
        scratch_types=[
            pltpu.VMEM((NCH, CHP), jnp.int32),
            pltpu.VMEM((NCH, CHP), jnp.int32),
            [pltpu.VMEM((CHP, H), jnp.float32) for _ in range(NB)],
            pltpu.VMEM((ZB, H), jnp.float32),
            pltpu.VMEM_SHARED((N + 8, H), jnp.float32),
            [pltpu.SemaphoreType.DMA for _ in range(NB)],
            [pltpu.SemaphoreType.DMA for _ in range(NB)],
        ],
    )
    def k(hsa_hbm, hsb_hbm, src_hbm, dst_hbm, zero_hbm, out_a, out_b,
          idx_s, idx_d, rows, zbuf, acc, gsem, ssem):
        cid = lax.axis_index("c")
        sid = lax.axis_index("s")
        with jax.named_scope("agg_init"):
            pltpu.sync_copy(zero_hbm, zbuf)
            for kk in range(RB // ZB):
                pltpu.sync_copy(zbuf, acc.at[pl.ds(sid * RB + kk * ZB, ZB)])

            @pl.when(sid == NS - 1)
            def _():
                pltpu.sync_copy(zbuf.at[pl.ds(0, 16)], acc.at[pl.ds(NS * RB, 16)])

            pltpu.sync_copy(src_hbm.at[sid], idx_s)
            pltpu.sync_copy(dst_hbm.at[sid], idx_d)
            plsc.subcore_barrier()

        hsp = [hsa_hbm, hsb_hbm]
        for c in range(NC):

            @pl.when(cid == c)
            def _(c=c):
                hsrc = hsp[c]
                for m in range(NB):
                    pltpu.async_copy(hsrc.at[idx_s.at[m]], rows[m], gsem[m])

                def group(gi, carry):
                    for b in range(NB):
                        j = gi * NB + b
                        pltpu.make_async_copy(
                            hsrc.at[idx_s.at[j]], rows[b], gsem[b]).wait()
                        pltpu.async_copy(
                            rows[b], acc.at[idx_d.at[j]], ssem[b], add=True)
                        bl = (b - L) % NB
                        jl = j - L

                        @pl.when(jl >= 0)
                        def _():
                            pltpu.make_async_copy(
                                rows[bl], acc.at[idx_d.at[jl]], ssem[bl]).wait()

                            @pl.when(jl + NB < NCH)
                            def _():
                                pltpu.async_copy(
                                    hsrc.at[idx_s.at[jl + NB]], rows[bl],
                                    gsem[bl])

                    return carry

                with jax.named_scope("agg_loop"):
                    lax.fori_loop(0, NCH // NB, group, 0)
                    for i in range(L):
                        j = NCH - L + i
                        b = j % NB
                        pltpu.make_async_copy(
                            rows[b], acc.at[idx_d.at[j]], ssem[b]).wait()

        plsc.subcore_barrier()

        out = [out_a, out_b]
        for c in range(NC):

            @pl.when(cid == c)
            def _w(c=c):
              with jax.named_scope("agg_out"):
                for kk in range(RB // ZB):
                    pltpu.sync_copy(acc.at[pl.ds(sid * RB + kk * ZB, ZB)], zbuf)
                    pltpu.sync_copy(zbuf, out[c].at[pl.ds(sid * RB + kk * ZB, ZB)])

                @pl.when(sid == NS - 1)
                def _():
                    pltpu.sync_copy(acc.at[pl.ds(NS * RB, 16)], rows[0].at[pl.ds(0, 16)])
                    pltpu.sync_copy(rows[0].at[pl.ds(0, 16)], out[c].at[pl.ds(NS * RB, 16)])

    return k(hs_a, hs_b, src3, dst3, zeros2d)


def _dis(deg_ref):
    deg = deg_ref[:, 0:1] + deg_ref[:, 1:2] + 1.0
    return lax.rsqrt(deg)


def _mm1_body(x_ref, w_ref, deg_ref, hsa_ref, hsb_ref):
    h = jnp.dot(x_ref[...], w_ref[...], preferred_element_type=jnp.float32)
    hs = h * _dis(deg_ref)
    hsa_ref[...] = hs[:, :H]
    hsb_ref[...] = hs[:, H:]


def _mm2_body(aa_ref, ab_ref, hsa_ref, hsb_ref, deg_ref, b_ref, w_ref,
              oa_ref, ob_ref):
    dis = _dis(deg_ref)
    za = dis * (aa_ref[...] + hsa_ref[...]) + b_ref[:, :H]
    zb = dis * (ab_ref[...] + hsb_ref[...]) + b_ref[:, H:]
    h = jnp.maximum(jnp.concatenate([za, zb], axis=1), 0.0)
    hs2 = jnp.dot(h, w_ref[...], preferred_element_type=jnp.float32) * dis
    oa_ref[...] = hs2[:, :H]
    ob_ref[...] = hs2[:, H:]


def _fin_body(aa_ref, ab_ref, hsa_ref, hsb_ref, deg_ref, b_ref, out_ref):
    dis = _dis(deg_ref)
    za = dis * (aa_ref[...] + hsa_ref[...]) + b_ref[:, :H]
    zb = dis * (ab_ref[...] + hsb_ref[...]) + b_ref[:, H:]
    out_ref[...] = jnp.concatenate([za, zb], axis=1)


_row = pl.BlockSpec((BN, D), lambda i: (i, 0))
_half = pl.BlockSpec((BN, H), lambda i: (i, 0))
_deg_spec = pl.BlockSpec((BN, 2), lambda i: (i, 0))
_full = pl.BlockSpec((D, D), lambda i: (0, 0))
_bias = pl.BlockSpec((1, D), lambda i: (0, 0))
_G = N // BN
_half_out = jax.ShapeDtypeStruct((N, H), jnp.float32)


def _mm1_call(x, W1, degt):
    return pl.pallas_call(
        _mm1_body,
        grid=(_G,),
        in_specs=[_row, _full, _deg_spec],
        out_specs=(_half, _half),
        out_shape=(_half_out, _half_out),
    )(x, W1, degt)


def _mm2_call(agg_a, agg_b, hs1a, hs1b, degt, b1, W2):
    return pl.pallas_call(
        _mm2_body,
        grid=(_G,),
        in_specs=[_half, _half, _half, _half, _deg_spec, _bias, _full],
        out_specs=(_half, _half),
        out_shape=(_half_out, _half_out),
    )(agg_a, agg_b, hs1a, hs1b, degt, b1, W2)


def _fin_call(agg_a, agg_b, hs2a, hs2b, degt, b2):
    return pl.pallas_call(
        _fin_body,
        grid=(_G,),
        in_specs=[_half, _half, _half, _half, _deg_spec, _bias],
        out_specs=_row,
        out_shape=jax.ShapeDtypeStruct((N, D), jnp.float32),
    )(agg_a, agg_b, hs2a, hs2b, degt, b2)


def kernel(x, edge_index, W1, b1, W2, b2):
    src = edge_index[0].astype(jnp.int32)
    dst = edge_index[1].astype(jnp.int32)
    src_d = src.reshape(NW, NCHD, CH)
    dst_d = dst.reshape(NW, NCHD, CH)
    pad = NS * EPT - E
    src_s = jnp.concatenate([src, jnp.zeros((pad,), jnp.int32)]).reshape(
        NS, NCH, CHP)
    dst_s = jnp.concatenate([dst, jnp.full((pad,), N, jnp.int32)]).reshape(
        NS, NCH, CHP)
    zeros2d = jnp.zeros((ZB, H), jnp.float32)
    zeros1d = jnp.zeros((RB + 16,), jnp.float32)
    ones1d = jnp.ones((CH,), jnp.float32)

    deg_a, deg_b = _deg_call(dst_d, zeros1d, ones1d)
    degt = jnp.stack([deg_a, deg_b], axis=1)

    hs1a, hs1b = _mm1_call(x, W1, degt)
    agg1a, agg1b = _agg_call(hs1a, hs1b, src_s, dst_s, zeros2d)
    hs2a, hs2b = _mm2_call(agg1a, agg1b, hs1a, hs1b, degt, b1.reshape(1, D), W2)
    agg2a, agg2b = _agg_call(hs2a, hs2b, src_s, dst_s, zeros2d)
    return _fin_call(agg2a, agg2b, hs2a, hs2b, degt, b2.reshape(1, D))


# speedup vs baseline: 2.2544x; 2.2544x over previous
import functools

import jax
import jax.numpy as jnp
from jax import lax
from jax.experimental import pallas as pl
from jax.experimental.pallas import tpu as pltpu
from jax.experimental.pallas import tpu_sc as plsc

N = 10000
E = 320000
D = 128
H = D // 2
NC = 2
NS = 16
NW = NC * NS
CH = 80
NCHD = E // NW // CH
CHP = 128
EPT = 20480
NCH = EPT // CHP
NB = 5
L = 2
RB = 624
ZB = 104
BN = 1000


def _sc_mesh():
    return plsc.VectorSubcoreMesh(
        core_axis_name="c", subcore_axis_name="s", num_cores=NC, num_subcores=NS
    )


def _deg_call(dst2d, zeros1d, ones1d):
    @functools.partial(
        pl.kernel,
        out_type=(
            jax.ShapeDtypeStruct((N,), jnp.float32),
            jax.ShapeDtypeStruct((N,), jnp.float32),
        ),
        mesh=_sc_mesh(),
        scratch_types=[
            pltpu.VMEM((NCHD, CH), jnp.int32),
            pltpu.VMEM((CH,), jnp.float32),
            pltpu.VMEM((RB + 16,), jnp.float32),
            pltpu.VMEM_SHARED((N,), jnp.float32),
        ],
    )
    def k(dst_hbm, zero_hbm, ones_hbm, out_a, out_b, idx_d, ones_v, zbuf, acc):
        cid = lax.axis_index("c")
        sid = lax.axis_index("s")
        wid = cid * NS + sid
        pltpu.sync_copy(zero_hbm, zbuf)
        pltpu.sync_copy(zbuf.at[pl.ds(0, RB)], acc.at[pl.ds(sid * RB, RB)])

        @pl.when(sid == NS - 1)
        def _():
            pltpu.sync_copy(zbuf.at[pl.ds(0, 16)], acc.at[pl.ds(NS * RB, 16)])

        pltpu.sync_copy(ones_hbm, ones_v)
        pltpu.sync_copy(dst_hbm.at[wid], idx_d)
        plsc.subcore_barrier()

        def body(j, carry):
            pltpu.sync_copy(ones_v, acc.at[idx_d.at[j]], add=True)
            return carry

        lax.fori_loop(0, NCHD, body, 0)
        plsc.subcore_barrier()

        pltpu.sync_copy(acc.at[pl.ds(sid * RB, RB)], zbuf.at[pl.ds(0, RB)])
        out = [out_a, out_b]
        for c in range(NC):

            @pl.when(cid == c)
            def _(c=c):
                pltpu.sync_copy(zbuf.at[pl.ds(0, RB)], out[c].at[pl.ds(sid * RB, RB)])

                @pl.when(sid == NS - 1)
                def _():
                    pltpu.sync_copy(acc.at[pl.ds(NS * RB, 16)], zbuf.at[pl.ds(RB, 16)])
                    pltpu.sync_copy(zbuf.at[pl.ds(RB, 16)], out[c].at[pl.ds(NS * RB, 16)])

    return k(dst2d, zeros1d, ones1d)


def _agg_call(hs_a, hs_b, src3, dst3, zeros2d):
    @functools.partial(
        pl.kernel,
        out_type=(
            jax.ShapeDtypeStruct((N, H), jnp.float32),
            jax.ShapeDtypeStruct((N, H), jnp.float32),
        ),
        mesh=_sc_mesh(),
        compiler_params=pltpu.CompilerParams(use_tc_tiling_on_sc=False),
        scratch_types=[
            pltpu.VMEM((NCH, CHP), jnp.int32),
            pltpu.VMEM((NCH, CHP), jnp.int32),
            [pltpu.VMEM((CHP, H), jnp.float32) for _ in range(NB)],
            pltpu.VMEM((ZB, H), jnp.float32),
            pltpu.VMEM_SHARED((N + 8, H), jnp.float32),
            [pltpu.SemaphoreType.DMA for _ in range(NB)],
            [pltpu.SemaphoreType.DMA for _ in range(NB)],
        ],
    )
    def k(hsa_hbm, hsb_hbm, src_hbm, dst_hbm, zero_hbm, out_a, out_b,
          idx_s, idx_d, rows, zbuf, acc, gsem, ssem):
        cid = lax.axis_index("c")
        sid = lax.axis_index("s")
        with jax.named_scope("agg_init"):
            pltpu.sync_copy(zero_hbm, zbuf)
            for kk in range(RB // ZB):
                pltpu.sync_copy(zbuf, acc.at[pl.ds(sid * RB + kk * ZB, ZB)])

            @pl.when(sid == NS - 1)
            def _():
                pltpu.sync_copy(zbuf.at[pl.ds(0, 16)], acc.at[pl.ds(NS * RB, 16)])

            pltpu.sync_copy(src_hbm.at[sid], idx_s)
            pltpu.sync_copy(dst_hbm.at[sid], idx_d)
            plsc.subcore_barrier()

        hsp = [hsa_hbm, hsb_hbm]
        for c in range(NC):

            @pl.when(cid == c)
            def _(c=c):
                hsrc = hsp[c]
                for m in range(NB):
                    pltpu.async_copy(hsrc.at[idx_s.at[m]], rows[m], gsem[m])

                def group(gi, carry):
                    for b in range(NB):
                        j = gi * NB + b
                        pltpu.make_async_copy(
                            hsrc.at[idx_s.at[j]], rows[b], gsem[b]).wait()
                        pltpu.async_copy(
                            rows[b], acc.at[idx_d.at[j]], ssem[b], add=True)
                        bl = (b - L) % NB
                        jl = j - L

                        @pl.when(jl >= 0)
                        def _():
                            pltpu.make_async_copy(
                                rows[bl], acc.at[idx_d.at[jl]], ssem[bl]).wait()

                            @pl.when(jl + NB < NCH)
                            def _():
                                pltpu.async_copy(
                                    hsrc.at[idx_s.at[jl + NB]], rows[bl],
                                    gsem[bl])

                    return carry

                with jax.named_scope("agg_loop"):
                    lax.fori_loop(0, NCH // NB, group, 0)
                    for i in range(L):
                        j = NCH - L + i
                        b = j % NB
                        pltpu.make_async_copy(
                            rows[b], acc.at[idx_d.at[j]], ssem[b]).wait()

        plsc.subcore_barrier()

        out = [out_a, out_b]
        for c in range(NC):

            @pl.when(cid == c)
            def _w(c=c):
              with jax.named_scope("agg_out"):
                for kk in range(RB // ZB):
                    pltpu.sync_copy(acc.at[pl.ds(sid * RB + kk * ZB, ZB)], zbuf)
                    pltpu.sync_copy(zbuf, out[c].at[pl.ds(sid * RB + kk * ZB, ZB)])

                @pl.when(sid == NS - 1)
                def _():
                    pltpu.sync_copy(acc.at[pl.ds(NS * RB, 16)], rows[0].at[pl.ds(0, 16)])
                    pltpu.sync_copy(rows[0].at[pl.ds(0, 16)], out[c].at[pl.ds(NS * RB, 16)])

    return k(hs_a, hs_b, src3, dst3, zeros2d)


def _dis(deg_ref):
    deg = deg_ref[:, 0:1] + deg_ref[:, 1:2] + 1.0
    return lax.rsqrt(deg)


def _mm1_body(x_ref, w_ref, deg_ref, hsa_ref, hsb_ref):
    h = jnp.dot(x_ref[...], w_ref[...], preferred_element_type=jnp.float32)
    hs = h * _dis(deg_ref)
    hsa_ref[...] = hs[:, :H]
    hsb_ref[...] = hs[:, H:]


def _mm2_body(aa_ref, ab_ref, hsa_ref, hsb_ref, deg_ref, b_ref, w_ref,
              oa_ref, ob_ref):
    dis = _dis(deg_ref)
    za = dis * (aa_ref[...] + hsa_ref[...]) + b_ref[:, :H]
    zb = dis * (ab_ref[...] + hsb_ref[...]) + b_ref[:, H:]
    h = jnp.maximum(jnp.concatenate([za, zb], axis=1), 0.0)
    hs2 = jnp.dot(h, w_ref[...], preferred_element_type=jnp.float32) * dis
    oa_ref[...] = hs2[:, :H]
    ob_ref[...] = hs2[:, H:]


def _fin_body(aa_ref, ab_ref, hsa_ref, hsb_ref, deg_ref, b_ref, out_ref):
    dis = _dis(deg_ref)
    za = dis * (aa_ref[...] + hsa_ref[...]) + b_ref[:, :H]
    zb = dis * (ab_ref[...] + hsb_ref[...]) + b_ref[:, H:]
    out_ref[...] = jnp.concatenate([za, zb], axis=1)


_row = pl.BlockSpec((BN, D), lambda i: (i, 0))
_half = pl.BlockSpec((BN, H), lambda i: (i, 0))
_deg_spec = pl.BlockSpec((BN, 2), lambda i: (i, 0))
_full = pl.BlockSpec((D, D), lambda i: (0, 0))
_bias = pl.BlockSpec((1, D), lambda i: (0, 0))
_G = N // BN
_half_out = jax.ShapeDtypeStruct((N, H), jnp.float32)


def _mm1_call(x, W1, degt):
    return pl.pallas_call(
        _mm1_body,
        grid=(_G,),
        in_specs=[_row, _full, _deg_spec],
        out_specs=(_half, _half),
        out_shape=(_half_out, _half_out),
    )(x, W1, degt)


def _mm2_call(agg_a, agg_b, hs1a, hs1b, degt, b1, W2):
    return pl.pallas_call(
        _mm2_body,
        grid=(_G,),
        in_specs=[_half, _half, _half, _half, _deg_spec, _bias, _full],
        out_specs=(_half, _half),
        out_shape=(_half_out, _half_out),
    )(agg_a, agg_b, hs1a, hs1b, degt, b1, W2)


def _fin_call(agg_a, agg_b, hs2a, hs2b, degt, b2):
    return pl.pallas_call(
        _fin_body,
        grid=(_G,),
        in_specs=[_half, _half, _half, _half, _deg_spec, _bias],
        out_specs=_row,
        out_shape=jax.ShapeDtypeStruct((N, D), jnp.float32),
    )(agg_a, agg_b, hs2a, hs2b, degt, b2)


def kernel(x, edge_index, W1, b1, W2, b2):
    src = edge_index[0].astype(jnp.int32)
    dst = edge_index[1].astype(jnp.int32)
    src_d = src.reshape(NW, NCHD, CH)
    dst_d = dst.reshape(NW, NCHD, CH)
    pad = EPT - E // NS
    srcr = src.reshape(NS, E // NS)
    dstr = dst.reshape(NS, E // NS)
    pad_src = jnp.broadcast_to(
        (jnp.arange(pad, dtype=jnp.int32) * (N // pad)), (NS, pad))
    pad_dst = (N + (jnp.arange(NS, dtype=jnp.int32)[:, None]
                    + jnp.arange(pad, dtype=jnp.int32)[None, :]) % 8)
    src_s = jnp.concatenate([srcr, pad_src], axis=1).reshape(NS, NCH, CHP)
    dst_s = jnp.concatenate([dstr, pad_dst], axis=1).reshape(NS, NCH, CHP)
    zeros2d = jnp.zeros((ZB, H), jnp.float32)
    zeros1d = jnp.zeros((RB + 16,), jnp.float32)
    ones1d = jnp.ones((CH,), jnp.float32)

    deg_a, deg_b = _deg_call(dst_d, zeros1d, ones1d)
    degt = jnp.stack([deg_a, deg_b], axis=1)

    hs1a, hs1b = _mm1_call(x, W1, degt)
    agg1a, agg1b = _agg_call(hs1a, hs1b, src_s, dst_s, zeros2d)
    hs2a, hs2b = _mm2_call(agg1a, agg1b, hs1a, hs1b, degt, b1.reshape(1, D), W2)
    agg2a, agg2b = _agg_call(hs2a, hs2b, src_s, dst_s, zeros2d)
    return _fin_call(agg2a, agg2b, hs2a, hs2b, degt, b2.reshape(1, D))


# speedup vs baseline: 2.2837x; 1.0130x over previous
import functools

import jax
import jax.numpy as jnp
from jax import lax
from jax.experimental import pallas as pl
from jax.experimental.pallas import tpu as pltpu
from jax.experimental.pallas import tpu_sc as plsc

N = 10000
E = 320000
D = 128
H = D // 2
NC = 2
NS = 16
NW = NC * NS
CH = 80
NCHD = E // NW // CH
CHP = 128
EPT = 20480
NCH = EPT // CHP
NB = 5
L = 2
RB = 624
ZB = 104
BN = 2000


def _sc_mesh():
    return plsc.VectorSubcoreMesh(
        core_axis_name="c", subcore_axis_name="s", num_cores=NC, num_subcores=NS
    )


def _deg_call(dst2d, zeros1d, ones1d):
    @functools.partial(
        pl.kernel,
        out_type=(
            jax.ShapeDtypeStruct((N,), jnp.float32),
            jax.ShapeDtypeStruct((N,), jnp.float32),
        ),
        mesh=_sc_mesh(),
        scratch_types=[
            pltpu.VMEM((NCHD, CH), jnp.int32),
            pltpu.VMEM((CH,), jnp.float32),
            pltpu.VMEM((RB + 16,), jnp.float32),
            pltpu.VMEM_SHARED((N,), jnp.float32),
        ],
    )
    def k(dst_hbm, zero_hbm, ones_hbm, out_a, out_b, idx_d, ones_v, zbuf, acc):
        cid = lax.axis_index("c")
        sid = lax.axis_index("s")
        wid = cid * NS + sid
        pltpu.sync_copy(zero_hbm, zbuf)
        pltpu.sync_copy(zbuf.at[pl.ds(0, RB)], acc.at[pl.ds(sid * RB, RB)])

        @pl.when(sid == NS - 1)
        def _():
            pltpu.sync_copy(zbuf.at[pl.ds(0, 16)], acc.at[pl.ds(NS * RB, 16)])

        pltpu.sync_copy(ones_hbm, ones_v)
        pltpu.sync_copy(dst_hbm.at[wid], idx_d)
        plsc.subcore_barrier()

        def body(j, carry):
            pltpu.sync_copy(ones_v, acc.at[idx_d.at[j]], add=True)
            return carry

        lax.fori_loop(0, NCHD, body, 0)
        plsc.subcore_barrier()

        pltpu.sync_copy(acc.at[pl.ds(sid * RB, RB)], zbuf.at[pl.ds(0, RB)])
        out = [out_a, out_b]
        for c in range(NC):

            @pl.when(cid == c)
            def _(c=c):
                pltpu.sync_copy(zbuf.at[pl.ds(0, RB)], out[c].at[pl.ds(sid * RB, RB)])

                @pl.when(sid == NS - 1)
                def _():
                    pltpu.sync_copy(acc.at[pl.ds(NS * RB, 16)], zbuf.at[pl.ds(RB, 16)])
                    pltpu.sync_copy(zbuf.at[pl.ds(RB, 16)], out[c].at[pl.ds(NS * RB, 16)])

    return k(dst2d, zeros1d, ones1d)


def _agg_call(hs_a, hs_b, src3, dst3, zeros2d):
    @functools.partial(
        pl.kernel,
        out_type=(
            jax.ShapeDtypeStruct((N, H), jnp.float32),
            jax.ShapeDtypeStruct((N, H), jnp.float32),
        ),
        mesh=_sc_mesh(),
        compiler_params=pltpu.CompilerParams(use_tc_tiling_on_sc=False),
        scratch_types=[
            pltpu.VMEM((NCH, CHP), jnp.int32),
            pltpu.VMEM((NCH, CHP), jnp.int32),
            [pltpu.VMEM((CHP, H), jnp.float32) for _ in range(NB)],
            pltpu.VMEM((ZB, H), jnp.float32),
            pltpu.VMEM_SHARED((N + 8, H), jnp.float32),
            [pltpu.SemaphoreType.DMA for _ in range(NB)],
            [pltpu.SemaphoreType.DMA for _ in range(NB)],
        ],
    )
    def k(hsa_hbm, hsb_hbm, src_hbm, dst_hbm, zero_hbm, out_a, out_b,
          idx_s, idx_d, rows, zbuf, acc, gsem, ssem):
        cid = lax.axis_index("c")
        sid = lax.axis_index("s")
        with jax.named_scope("agg_init"):
            pltpu.sync_copy(zero_hbm, zbuf)
            for kk in range(RB // ZB):
                pltpu.sync_copy(zbuf, acc.at[pl.ds(sid * RB + kk * ZB, ZB)])

            @pl.when(sid == NS - 1)
            def _():
                pltpu.sync_copy(zbuf.at[pl.ds(0, 16)], acc.at[pl.ds(NS * RB, 16)])

            pltpu.sync_copy(src_hbm.at[sid], idx_s)
            pltpu.sync_copy(dst_hbm.at[sid], idx_d)
            plsc.subcore_barrier()

        hsp = [hsa_hbm, hsb_hbm]
        for c in range(NC):

            @pl.when(cid == c)
            def _(c=c):
                hsrc = hsp[c]
                for m in range(NB):
                    pltpu.async_copy(hsrc.at[idx_s.at[m]], rows[m], gsem[m])

                def group(gi, carry):
                    for b in range(NB):
                        j = gi * NB + b
                        pltpu.make_async_copy(
                            hsrc.at[idx_s.at[j]], rows[b], gsem[b]).wait()
                        pltpu.async_copy(
                            rows[b], acc.at[idx_d.at[j]], ssem[b], add=True)
                        bl = (b - L) % NB
                        jl = j - L

                        @pl.when(jl >= 0)
                        def _():
                            pltpu.make_async_copy(
                                rows[bl], acc.at[idx_d.at[jl]], ssem[bl]).wait()

                            @pl.when(jl + NB < NCH)
                            def _():
                                pltpu.async_copy(
                                    hsrc.at[idx_s.at[jl + NB]], rows[bl],
                                    gsem[bl])

                    return carry

                with jax.named_scope("agg_loop"):
                    lax.fori_loop(0, NCH // NB, group, 0)
                    for i in range(L):
                        j = NCH - L + i
                        b = j % NB
                        pltpu.make_async_copy(
                            rows[b], acc.at[idx_d.at[j]], ssem[b]).wait()

        plsc.subcore_barrier()

        out = [out_a, out_b]
        for c in range(NC):

            @pl.when(cid == c)
            def _w(c=c):
              with jax.named_scope("agg_out"):
                for kk in range(RB // ZB):
                    pltpu.sync_copy(acc.at[pl.ds(sid * RB + kk * ZB, ZB)], zbuf)
                    pltpu.sync_copy(zbuf, out[c].at[pl.ds(sid * RB + kk * ZB, ZB)])

                @pl.when(sid == NS - 1)
                def _():
                    pltpu.sync_copy(acc.at[pl.ds(NS * RB, 16)], rows[0].at[pl.ds(0, 16)])
                    pltpu.sync_copy(rows[0].at[pl.ds(0, 16)], out[c].at[pl.ds(NS * RB, 16)])

    return k(hs_a, hs_b, src3, dst3, zeros2d)


def _dis(deg_ref):
    deg = deg_ref[:, 0:1] + deg_ref[:, 1:2] + 1.0
    return lax.rsqrt(deg)


def _mm1_body(x_ref, w_ref, deg_ref, hsa_ref, hsb_ref):
    h = jnp.dot(x_ref[...], w_ref[...], preferred_element_type=jnp.float32)
    hs = h * _dis(deg_ref)
    hsa_ref[...] = hs[:, :H]
    hsb_ref[...] = hs[:, H:]


def _mm2_body(aa_ref, ab_ref, hsa_ref, hsb_ref, deg_ref, b_ref, w_ref,
              oa_ref, ob_ref):
    dis = _dis(deg_ref)
    za = dis * (aa_ref[...] + hsa_ref[...]) + b_ref[:, :H]
    zb = dis * (ab_ref[...] + hsb_ref[...]) + b_ref[:, H:]
    h = jnp.maximum(jnp.concatenate([za, zb], axis=1), 0.0)
    hs2 = jnp.dot(h, w_ref[...], preferred_element_type=jnp.float32) * dis
    oa_ref[...] = hs2[:, :H]
    ob_ref[...] = hs2[:, H:]


def _fin_body(aa_ref, ab_ref, hsa_ref, hsb_ref, deg_ref, b_ref, out_ref):
    dis = _dis(deg_ref)
    za = dis * (aa_ref[...] + hsa_ref[...]) + b_ref[:, :H]
    zb = dis * (ab_ref[...] + hsb_ref[...]) + b_ref[:, H:]
    out_ref[...] = jnp.concatenate([za, zb], axis=1)


_row = pl.BlockSpec((BN, D), lambda i: (i, 0))
_half = pl.BlockSpec((BN, H), lambda i: (i, 0))
_deg_spec = pl.BlockSpec((BN, 2), lambda i: (i, 0))
_full = pl.BlockSpec((D, D), lambda i: (0, 0))
_bias = pl.BlockSpec((1, D), lambda i: (0, 0))
_G = N // BN
_half_out = jax.ShapeDtypeStruct((N, H), jnp.float32)


def _mm1_call(x, W1, degt):
    return pl.pallas_call(
        _mm1_body,
        grid=(_G,),
        in_specs=[_row, _full, _deg_spec],
        out_specs=(_half, _half),
        out_shape=(_half_out, _half_out),
    )(x, W1, degt)


def _mm2_call(agg_a, agg_b, hs1a, hs1b, degt, b1, W2):
    return pl.pallas_call(
        _mm2_body,
        grid=(_G,),
        in_specs=[_half, _half, _half, _half, _deg_spec, _bias, _full],
        out_specs=(_half, _half),
        out_shape=(_half_out, _half_out),
    )(agg_a, agg_b, hs1a, hs1b, degt, b1, W2)


def _fin_call(agg_a, agg_b, hs2a, hs2b, degt, b2):
    return pl.pallas_call(
        _fin_body,
        grid=(_G,),
        in_specs=[_half, _half, _half, _half, _deg_spec, _bias],
        out_specs=_row,
        out_shape=jax.ShapeDtypeStruct((N, D), jnp.float32),
    )(agg_a, agg_b, hs2a, hs2b, degt, b2)


def kernel(x, edge_index, W1, b1, W2, b2):
    src = edge_index[0].astype(jnp.int32)
    dst = edge_index[1].astype(jnp.int32)
    src_d = src.reshape(NW, NCHD, CH)
    dst_d = dst.reshape(NW, NCHD, CH)
    pad = EPT - E // NS
    srcr = src.reshape(NS, E // NS)
    dstr = dst.reshape(NS, E // NS)
    pad_src = jnp.broadcast_to(
        (jnp.arange(pad, dtype=jnp.int32) * (N // pad)), (NS, pad))
    pad_dst = (N + (jnp.arange(NS, dtype=jnp.int32)[:, None]
                    + jnp.arange(pad, dtype=jnp.int32)[None, :]) % 8)
    src_s = jnp.concatenate([srcr, pad_src], axis=1).reshape(NS, NCH, CHP)
    dst_s = jnp.concatenate([dstr, pad_dst], axis=1).reshape(NS, NCH, CHP)
    zeros2d = jnp.zeros((ZB, H), jnp.float32)
    zeros1d = jnp.zeros((RB + 16,), jnp.float32)
    ones1d = jnp.ones((CH,), jnp.float32)

    deg_a, deg_b = _deg_call(dst_d, zeros1d, ones1d)
    degt = jnp.stack([deg_a, deg_b], axis=1)

    hs1a, hs1b = _mm1_call(x, W1, degt)
    agg1a, agg1b = _agg_call(hs1a, hs1b, src_s, dst_s, zeros2d)
    hs2a, hs2b = _mm2_call(agg1a, agg1b, hs1a, hs1b, degt, b1.reshape(1, D), W2)
    agg2a, agg2b = _agg_call(hs2a, hs2b, src_s, dst_s, zeros2d)
    return _fin_call(agg2a, agg2b, hs2a, hs2b, degt, b2.reshape(1, D))


# speedup vs baseline: 2.5548x; 1.1187x over previous
import functools

import jax
import jax.numpy as jnp
from jax import lax
from jax.experimental import pallas as pl
from jax.experimental.pallas import tpu as pltpu
from jax.experimental.pallas import tpu_sc as plsc

N = 10000
E = 320000
D = 128
H = D // 2
NC = 2
NS = 16
NW = NC * NS
CH = 80
NCHD = E // NW // CH
CHP = 128
EPT = 20480
NCH = EPT // CHP
NB = 5
L = 2
RB = 624
ZB = 104
BN = 2000


def _sc_mesh():
    return plsc.VectorSubcoreMesh(
        core_axis_name="c", subcore_axis_name="s", num_cores=NC, num_subcores=NS
    )


def _deg_call(dst2d, zeros1d, ones1d):
    @functools.partial(
        pl.kernel,
        out_type=(
            jax.ShapeDtypeStruct((N,), jnp.float32),
            jax.ShapeDtypeStruct((N,), jnp.float32),
        ),
        mesh=_sc_mesh(),
        scratch_types=[
            pltpu.VMEM((NCHD, CH), jnp.int32),
            pltpu.VMEM((CH,), jnp.float32),
            pltpu.VMEM((RB + 16,), jnp.float32),
            pltpu.VMEM_SHARED((N,), jnp.float32),
        ],
    )
    def k(dst_hbm, zero_hbm, ones_hbm, out_a, out_b, idx_d, ones_v, zbuf, acc):
        cid = lax.axis_index("c")
        sid = lax.axis_index("s")
        wid = cid * NS + sid
        pltpu.sync_copy(zero_hbm, zbuf)
        pltpu.sync_copy(zbuf.at[pl.ds(0, RB)], acc.at[pl.ds(sid * RB, RB)])

        @pl.when(sid == NS - 1)
        def _():
            pltpu.sync_copy(zbuf.at[pl.ds(0, 16)], acc.at[pl.ds(NS * RB, 16)])

        pltpu.sync_copy(ones_hbm, ones_v)
        pltpu.sync_copy(dst_hbm.at[wid], idx_d)
        plsc.subcore_barrier()

        def body(j, carry):
            pltpu.sync_copy(ones_v, acc.at[idx_d.at[j]], add=True)
            return carry

        lax.fori_loop(0, NCHD, body, 0)
        plsc.subcore_barrier()

        pltpu.sync_copy(acc.at[pl.ds(sid * RB, RB)], zbuf.at[pl.ds(0, RB)])
        out = [out_a, out_b]
        for c in range(NC):

            @pl.when(cid == c)
            def _(c=c):
                pltpu.sync_copy(zbuf.at[pl.ds(0, RB)], out[c].at[pl.ds(sid * RB, RB)])

                @pl.when(sid == NS - 1)
                def _():
                    pltpu.sync_copy(acc.at[pl.ds(NS * RB, 16)], zbuf.at[pl.ds(RB, 16)])
                    pltpu.sync_copy(zbuf.at[pl.ds(RB, 16)], out[c].at[pl.ds(NS * RB, 16)])

    return k(dst2d, zeros1d, ones1d)


def _agg_call(hs_a, hs_b, src3, dst3, zeros2d):
    @functools.partial(
        pl.kernel,
        out_type=(
            jax.ShapeDtypeStruct((N, H), jnp.float32),
            jax.ShapeDtypeStruct((N, H), jnp.float32),
        ),
        mesh=_sc_mesh(),
        compiler_params=pltpu.CompilerParams(use_tc_tiling_on_sc=False),
        scratch_types=[
            pltpu.VMEM((NCH, CHP), jnp.int32),
            pltpu.VMEM((NCH, CHP), jnp.int32),
            [pltpu.VMEM((CHP, H), jnp.float32) for _ in range(NB)],
            pltpu.VMEM((ZB, H), jnp.float32),
            pltpu.VMEM_SHARED((N + 8, H), jnp.float32),
            [pltpu.SemaphoreType.DMA for _ in range(NB)],
            [pltpu.SemaphoreType.DMA for _ in range(NB)],
        ],
    )
    def k(hsa_hbm, hsb_hbm, src_hbm, dst_hbm, zero_hbm, out_a, out_b,
          idx_s, idx_d, rows, zbuf, acc, gsem, ssem):
        cid = lax.axis_index("c")
        sid = lax.axis_index("s")
        with jax.named_scope("agg_init"):
            pltpu.sync_copy(zero_hbm, zbuf)
            for kk in range(RB // ZB):
                pltpu.sync_copy(zbuf, acc.at[pl.ds(sid * RB + kk * ZB, ZB)])

            @pl.when(sid == NS - 1)
            def _():
                pltpu.sync_copy(zbuf.at[pl.ds(0, 16)], acc.at[pl.ds(NS * RB, 16)])

            pltpu.sync_copy(src_hbm.at[sid], idx_s)
            pltpu.sync_copy(dst_hbm.at[sid], idx_d)
            plsc.subcore_barrier()

        hsp = [hsa_hbm, hsb_hbm]
        for c in range(NC):

            @pl.when(cid == c)
            def _(c=c):
                hsrc = hsp[c]
                for m in range(NB):
                    pltpu.async_copy(hsrc.at[idx_s.at[m]], rows[m], gsem[m])

                def group(gi, carry):
                    for b in range(NB):
                        j = gi * NB + b
                        pltpu.make_async_copy(
                            hsrc.at[idx_s.at[j]], rows[b], gsem[b]).wait()
                        pltpu.async_copy(
                            rows[b], acc.at[idx_d.at[j]], ssem[b], add=True)
                        bl = (b - L) % NB
                        jl = j - L

                        @pl.when(jl >= 0)
                        def _():
                            pltpu.make_async_copy(
                                rows[bl], acc.at[idx_d.at[jl]], ssem[bl]).wait()

                            @pl.when(jl + NB < NCH)
                            def _():
                                pltpu.async_copy(
                                    hsrc.at[idx_s.at[jl + NB]], rows[bl],
                                    gsem[bl])

                    return carry

                with jax.named_scope("agg_loop"):
                    lax.fori_loop(0, NCH // NB, group, 0)
                    for i in range(L):
                        j = NCH - L + i
                        b = j % NB
                        pltpu.make_async_copy(
                            rows[b], acc.at[idx_d.at[j]], ssem[b]).wait()

        plsc.subcore_barrier()

        out = [out_a, out_b]
        for c in range(NC):

            @pl.when(cid == c)
            def _w(c=c):
              with jax.named_scope("agg_out"):
                for kk in range(RB // ZB):
                    pltpu.sync_copy(acc.at[pl.ds(sid * RB + kk * ZB, ZB)], zbuf)
                    pltpu.sync_copy(zbuf, out[c].at[pl.ds(sid * RB + kk * ZB, ZB)])

                @pl.when(sid == NS - 1)
                def _():
                    pltpu.sync_copy(acc.at[pl.ds(NS * RB, 16)], rows[0].at[pl.ds(0, 16)])
                    pltpu.sync_copy(rows[0].at[pl.ds(0, 16)], out[c].at[pl.ds(NS * RB, 16)])

    return k(hs_a, hs_b, src3, dst3, zeros2d)


def _dis(deg_ref):
    deg = deg_ref[:, 0:1] + deg_ref[:, 1:2] + 1.0
    return lax.rsqrt(deg)


def _dis_eo(degp_ref):
    d = degp_ref[...]
    dis_e = lax.rsqrt(d[:, 0:1] + d[:, 1:2] + 1.0)
    dis_o = lax.rsqrt(d[:, 2:3] + d[:, 3:4] + 1.0)
    return dis_e, dis_o


def _dis128(dis_e, dis_o):
    lane = lax.broadcasted_iota(jnp.int32, (dis_e.shape[0], D), 1)
    return jnp.where(lane < H, dis_e, dis_o)


def _mm1_body(xp_ref, w_ref, degp_ref, pa_ref, pb_ref):
    w = w_ref[...]
    he = jnp.dot(xp_ref[:, :D], w, preferred_element_type=jnp.float32)
    ho = jnp.dot(xp_ref[:, D:], w, preferred_element_type=jnp.float32)
    dis_e, dis_o = _dis_eo(degp_ref)
    hse = he * dis_e
    hso = ho * dis_o
    pa_ref[...] = jnp.concatenate([hse[:, :H], hso[:, :H]], 1)
    pb_ref[...] = jnp.concatenate([hse[:, H:], hso[:, H:]], 1)


def _mm2_body(aa_ref, ab_ref, hsa_ref, hsb_ref, degp_ref, b_ref, w_ref,
              oa_ref, ob_ref):
    dis_e, dis_o = _dis_eo(degp_ref)
    dis128 = _dis128(dis_e, dis_o)
    b = b_ref[...]
    za = dis128 * (aa_ref[...] + hsa_ref[...]) + jnp.concatenate(
        [b[:, :H], b[:, :H]], 1)
    zb = dis128 * (ab_ref[...] + hsb_ref[...]) + jnp.concatenate(
        [b[:, H:], b[:, H:]], 1)
    pa = jnp.maximum(za, 0.0)
    pb = jnp.maximum(zb, 0.0)
    xe = jnp.concatenate([pa[:, :H], pb[:, :H]], 1)
    xo = jnp.concatenate([pa[:, H:], pb[:, H:]], 1)
    w = w_ref[...]
    ye = jnp.dot(xe, w, preferred_element_type=jnp.float32) * dis_e
    yo = jnp.dot(xo, w, preferred_element_type=jnp.float32) * dis_o
    oa_ref[...] = jnp.concatenate([ye[:, :H], yo[:, :H]], 1)
    ob_ref[...] = jnp.concatenate([ye[:, H:], yo[:, H:]], 1)


def _fin_body(aa_ref, ab_ref, hsa_ref, hsb_ref, degp_ref, b_ref, out_ref):
    dis_e, dis_o = _dis_eo(degp_ref)
    dis128 = _dis128(dis_e, dis_o)
    b = b_ref[...]
    za = dis128 * (aa_ref[...] + hsa_ref[...]) + jnp.concatenate(
        [b[:, :H], b[:, :H]], 1)
    zb = dis128 * (ab_ref[...] + hsb_ref[...]) + jnp.concatenate(
        [b[:, H:], b[:, H:]], 1)
    xe = jnp.concatenate([za[:, :H], zb[:, :H]], 1)
    xo = jnp.concatenate([za[:, H:], zb[:, H:]], 1)
    out_ref[...] = jnp.concatenate([xe, xo], 1)


N2 = N // 2
BN2 = BN // 2
_packed = pl.BlockSpec((BN2, D), lambda i: (i, 0))
_xp_spec = pl.BlockSpec((BN2, 2 * D), lambda i: (i, 0))
_degp_spec = pl.BlockSpec((BN2, 4), lambda i: (i, 0))
_full = pl.BlockSpec((D, D), lambda i: (0, 0))
_bias = pl.BlockSpec((1, D), lambda i: (0, 0))
_G = N // BN
_packed_out = jax.ShapeDtypeStruct((N2, D), jnp.float32)


def _mm1_call(xp, W1, degp):
    return pl.pallas_call(
        _mm1_body,
        grid=(_G,),
        in_specs=[_xp_spec, _full, _degp_spec],
        out_specs=(_packed, _packed),
        out_shape=(_packed_out, _packed_out),
    )(xp, W1, degp)


def _mm2_call(agg_a, agg_b, hs1a, hs1b, degp, b1, W2):
    return pl.pallas_call(
        _mm2_body,
        grid=(_G,),
        in_specs=[_packed, _packed, _packed, _packed, _degp_spec, _bias, _full],
        out_specs=(_packed, _packed),
        out_shape=(_packed_out, _packed_out),
    )(agg_a, agg_b, hs1a, hs1b, degp, b1, W2)


def _fin_call(agg_a, agg_b, hs2a, hs2b, degp, b2):
    return pl.pallas_call(
        _fin_body,
        grid=(_G,),
        in_specs=[_packed, _packed, _packed, _packed, _degp_spec, _bias],
        out_specs=_xp_spec,
        out_shape=jax.ShapeDtypeStruct((N2, 2 * D), jnp.float32),
    )(agg_a, agg_b, hs2a, hs2b, degp, b2)


def kernel(x, edge_index, W1, b1, W2, b2):
    src = edge_index[0].astype(jnp.int32)
    dst = edge_index[1].astype(jnp.int32)
    src_d = src.reshape(NW, NCHD, CH)
    dst_d = dst.reshape(NW, NCHD, CH)
    pad = EPT - E // NS
    srcr = src.reshape(NS, E // NS)
    dstr = dst.reshape(NS, E // NS)
    pad_src = jnp.broadcast_to(
        (jnp.arange(pad, dtype=jnp.int32) * (N // pad)), (NS, pad))
    pad_dst = (N + (jnp.arange(NS, dtype=jnp.int32)[:, None]
                    + jnp.arange(pad, dtype=jnp.int32)[None, :]) % 8)
    src_s = jnp.concatenate([srcr, pad_src], axis=1).reshape(NS, NCH, CHP)
    dst_s = jnp.concatenate([dstr, pad_dst], axis=1).reshape(NS, NCH, CHP)
    zeros2d = jnp.zeros((ZB, H), jnp.float32)
    zeros1d = jnp.zeros((RB + 16,), jnp.float32)
    ones1d = jnp.ones((CH,), jnp.float32)

    deg_a, deg_b = _deg_call(dst_d, zeros1d, ones1d)
    degp = jnp.stack([deg_a, deg_b], axis=1).reshape(N2, 4)

    hs1a_p, hs1b_p = _mm1_call(x.reshape(N2, 2 * D), W1, degp)
    agg1a, agg1b = _agg_call(
        hs1a_p.reshape(N, H), hs1b_p.reshape(N, H), src_s, dst_s, zeros2d)
    hs2a_p, hs2b_p = _mm2_call(
        agg1a.reshape(N2, D), agg1b.reshape(N2, D), hs1a_p, hs1b_p, degp,
        b1.reshape(1, D), W2)
    agg2a, agg2b = _agg_call(
        hs2a_p.reshape(N, H), hs2b_p.reshape(N, H), src_s, dst_s, zeros2d)
    return _fin_call(
        agg2a.reshape(N2, D), agg2b.reshape(N2, D), hs2a_p, hs2b_p, degp,
        b2.reshape(1, D)).reshape(N, D)
